# baseline (speedup 1.0000x reference)
import functools

import jax
import jax.numpy as jnp
from jax import lax
from jax.experimental import pallas as pl
from jax.experimental.pallas import tpu as pltpu
from jax.experimental.pallas import tpu_sc as plsc

B = 64
N = 32768
D = 8
KTOP = 3
NC, NS, L = 2, 16, 16
NW = NC * NS
BPW = B // NW
ROW = N * D
CHUNK = 16384
NCHUNK = (BPW * ROW) // CHUNK
CHUNKS_PER_BATCH = ROW // CHUNK
VPC = CHUNK // L
UNROLL = 8


def _insert3(m1, m2, m3, v):
    t1 = jnp.minimum(m1, v)
    m1 = jnp.maximum(m1, v)
    t2 = jnp.minimum(m2, t1)
    m2 = jnp.maximum(m2, t1)
    m3 = jnp.maximum(m3, t2)
    return m1, m2, m3


def _worker_id():
    return lax.axis_index("s") * NC + lax.axis_index("c")


def _body(x_hbm, coef_hbm, out_hbm, buf_a, buf_b, coef_v, out_v,
          sem_a, sem_b, sem_c):
    wid = _worker_id()
    base = wid * (BPW * ROW)

    pltpu.async_copy(coef_hbm, coef_v, sem_c).wait()

    bufs = (buf_a, buf_b)
    sems = (sem_a, sem_b)

    def start(ci):
        return pltpu.async_copy(
            x_hbm.at[pl.ds(base + ci * CHUNK, CHUNK)],
            bufs[ci % 2], sems[ci % 2])

    handles = {0: start(0), 1: start(1)}

    neg_inf = jnp.full((L,), -jnp.inf, dtype=jnp.float32)
    acc = [(neg_inf, neg_inf, neg_inf) for _ in range(BPW)]

    def make_chunk_body(buf):
        def chunk_body(i, carry):
            m1, m2, m3 = carry
            off = i * (L * UNROLL)
            for u in range(UNROLL):
                v = buf[pl.ds(off + u * L, L)]
                m1, m2, m3 = _insert3(m1, m2, m3, v)
            return m1, m2, m3
        return chunk_body

    chunk_bodies = (make_chunk_body(buf_a), make_chunk_body(buf_b))

    for ci in range(NCHUNK):
        lb = ci // CHUNKS_PER_BATCH
        handles[ci].wait()
        acc[lb] = lax.fori_loop(0, VPC // UNROLL, chunk_bodies[ci % 2],
                                acc[lb])
        if ci + 2 < NCHUNK:
            handles[ci + 2] = start(ci + 2)

    lanes = lax.iota(jnp.int32, L)
    idx_hi = jnp.bitwise_and(lanes, 7) + 8

    e1 = coef_v[0]
    e2 = coef_v[1]
    e3 = coef_v[2]
    bias = coef_v[3]

    res = []
    for lb in range(BPW):
        m1, m2, m3 = acc[lb]
        h1 = jnp.take_along_axis(m1, idx_hi, axis=0)
        h2 = jnp.take_along_axis(m2, idx_hi, axis=0)
        h3 = jnp.take_along_axis(m3, idx_hi, axis=0)
        m1, m2, m3 = _insert3(m1, m2, m3, h1)
        t = jnp.minimum(m2, h2)
        m2 = jnp.maximum(m2, h2)
        m3 = jnp.maximum(m3, jnp.maximum(t, h3))
        s = m1 * e1 + m2 * e2 + m3 * e3
        for sh in (4, 2, 1):
            s = s + jnp.take_along_axis(s, jnp.bitwise_xor(lanes, sh),
                                        axis=0)
        res.append(s)

    zero = jnp.zeros((L,), jnp.float32)
    ov = jnp.where(lanes == 0, res[0], zero) + \
         jnp.where(lanes == 1, res[1], zero) + bias
    out_v[:] = ov
    pltpu.async_copy(out_v, out_hbm.at[wid], sem_c).wait()


@jax.jit
def _launch(x_flat, coef):
    mesh = plsc.VectorSubcoreMesh(core_axis_name="c", subcore_axis_name="s",
                                  num_cores=NC, num_subcores=NS)
    return pl.kernel(
        _body,
        out_type=jax.ShapeDtypeStruct((NW, L), jnp.float32),
        mesh=mesh,
        scratch_types=[
            pltpu.VMEM((CHUNK,), jnp.float32),
            pltpu.VMEM((CHUNK,), jnp.float32),
            pltpu.VMEM((4, L), jnp.float32),
            pltpu.VMEM((L,), jnp.float32),
            pltpu.SemaphoreType.DMA,
            pltpu.SemaphoreType.DMA,
            pltpu.SemaphoreType.DMA,
        ],
    )(x_flat, coef)


def _make_coef(W, b):
    W0 = W.reshape(D, KTOP + 1)
    e = W0[:, :KTOP] + W0[:, KTOP:] / 3.0
    e_pad = jnp.concatenate([e, jnp.zeros_like(e)], axis=0)
    return jnp.concatenate(
        [e_pad.T, jnp.full((1, L), b[0], jnp.float32)], axis=0)


def kernel(x, W, b):
    out = _launch(x.reshape(-1), _make_coef(W, b))
    return out[:, :BPW].reshape(B, 1)


# speedup vs baseline: 13.7636x; 13.7636x over previous
import functools

import jax
import jax.numpy as jnp
from jax import lax
from jax.experimental import pallas as pl
from jax.experimental.pallas import tpu as pltpu
from jax.experimental.pallas import tpu_sc as plsc

B = 64
N = 32768
D = 8
KTOP = 3
NC, NS, L = 2, 16, 16
NW = NC * NS
BPW = B // NW
ROW = N * D
RUN = 128
GROUP = RUN * D
CHUNK = 16384
GROUPS_PER_CHUNK = CHUNK // GROUP
NCHUNK = (BPW * ROW) // CHUNK
PAIRS = NCHUNK // 2
PAIRS_PER_BATCH = PAIRS // BPW


def _insert3(m1, m2, m3, v):
    t1 = jnp.minimum(m1, v)
    m1 = jnp.maximum(m1, v)
    t2 = jnp.minimum(m2, t1)
    m2 = jnp.maximum(m2, t1)
    m3 = jnp.maximum(m3, t2)
    return m1, m2, m3


def _merge_triples(m1, m2, m3, h1, h2, h3):
    t1 = jnp.minimum(m1, h1)
    m1 = jnp.maximum(m1, h1)
    t2 = jnp.minimum(m2, t1)
    m2 = jnp.maximum(m2, t1)
    m3 = jnp.maximum(m3, t2)
    t = jnp.minimum(m2, h2)
    m2 = jnp.maximum(m2, h2)
    m3 = jnp.maximum(m3, jnp.maximum(t, h3))
    return m1, m2, m3


def _bf16_rne(x):
    u = lax.bitcast_convert_type(x, jnp.int32)
    lsb = jnp.bitwise_and(jnp.right_shift(u, 16), 1)
    r = jnp.bitwise_and(u + 32767 + lsb, jnp.int32(-65536))
    return lax.bitcast_convert_type(r, jnp.float32)


def _worker_id():
    return lax.axis_index("s") * NC + lax.axis_index("c")


def _body(x_hbm, coef_hbm, out_hbm, buf_a, buf_b, save_v, coef_v, out_v,
          sem_a, sem_b, sem_c):
    wid = _worker_id()
    base = wid * (BPW * ROW)

    pltpu.async_copy(coef_hbm, coef_v, sem_c).wait()

    def dma_start(ci, buf, sem):
        pltpu.make_async_copy(
            x_hbm.at[pl.ds(base + ci * CHUNK, CHUNK)], buf, sem).start()

    def dma_wait(buf, sem):
        pltpu.make_async_copy(
            x_hbm.at[pl.ds(0, CHUNK)], buf, sem).wait()

    dma_start(0, buf_a, sem_a)
    dma_start(1, buf_b, sem_b)

    neg_inf = jnp.full((L,), -jnp.inf, dtype=jnp.float32)
    acc0 = tuple(neg_inf for _ in range(3 * D))

    def process(buf, acc):
        def gbody(i, acc):
            accl = list(acc)
            off = i * GROUP
            for c in range(D):
                m1, m2, m3 = accl[3 * c], accl[3 * c + 1], accl[3 * c + 2]
                for u in range(RUN // L):
                    v = buf[pl.ds(off + c * RUN + u * L, L)]
                    m1, m2, m3 = _insert3(m1, m2, m3, v)
                accl[3 * c], accl[3 * c + 1], accl[3 * c + 2] = m1, m2, m3
            return tuple(accl)
        return lax.fori_loop(0, GROUPS_PER_CHUNK, gbody, acc)

    def pair_body(p, acc):
        @pl.when(p == PAIRS_PER_BATCH)
        def _():
            for k in range(3 * D):
                save_v[pl.ds(k * L, L)] = acc[k]
        reset = p == PAIRS_PER_BATCH
        acc = tuple(jnp.where(reset, neg_inf, a) for a in acc)

        ci = p * 2
        dma_wait(buf_a, sem_a)
        acc = process(buf_a, acc)

        @pl.when(ci + 2 < NCHUNK)
        def _():
            dma_start(ci + 2, buf_a, sem_a)

        dma_wait(buf_b, sem_b)
        acc = process(buf_b, acc)

        @pl.when(ci + 3 < NCHUNK)
        def _():
            dma_start(ci + 3, buf_b, sem_b)

        return acc

    acc = lax.fori_loop(0, PAIRS, pair_body, acc0)

    lanes = lax.iota(jnp.int32, L)

    def top3_all_lanes(m1, m2, m3):
        for sh in (8, 4, 2, 1):
            idx = jnp.bitwise_xor(lanes, sh)
            h1 = jnp.take_along_axis(m1, idx, axis=0)
            h2 = jnp.take_along_axis(m2, idx, axis=0)
            h3 = jnp.take_along_axis(m3, idx, axis=0)
            m1, m2, m3 = _merge_triples(m1, m2, m3, h1, h2, h3)
        return m1, m2, m3

    def batch_result(get):
        zero = jnp.zeros((L,), jnp.float32)
        M1 = M2 = M3 = zero
        for c in range(D):
            m1, m2, m3 = top3_all_lanes(*get(c))
            sel = lanes == c
            M1 = jnp.where(sel, m1, M1)
            M2 = jnp.where(sel, m2, M2)
            M3 = jnp.where(sel, m3, M3)
        Fm = ((M1 + M2) + M3) / 3.0
        s = _bf16_rne(M1) * coef_v[0] + _bf16_rne(M2) * coef_v[1] + \
            _bf16_rne(M3) * coef_v[2] + _bf16_rne(Fm) * coef_v[3]
        for sh in (4, 2, 1):
            s = s + jnp.take_along_axis(s, jnp.bitwise_xor(lanes, sh),
                                        axis=0)
        return s

    s0 = batch_result(
        lambda c: (save_v[pl.ds((3 * c) * L, L)],
                   save_v[pl.ds((3 * c + 1) * L, L)],
                   save_v[pl.ds((3 * c + 2) * L, L)]))
    s1 = batch_result(
        lambda c: (acc[3 * c], acc[3 * c + 1], acc[3 * c + 2]))

    zero = jnp.zeros((L,), jnp.float32)
    ov = jnp.where(lanes == 0, s0, zero) + \
         jnp.where(lanes == 1, s1, zero) + coef_v[4]
    out_v[:] = ov
    pltpu.async_copy(out_v, out_hbm.at[wid], sem_c).wait()


@jax.jit
def _launch(x_flat, coef):
    mesh = plsc.VectorSubcoreMesh(core_axis_name="c", subcore_axis_name="s",
                                  num_cores=NC, num_subcores=NS)
    return pl.kernel(
        _body,
        out_type=jax.ShapeDtypeStruct((NW, L), jnp.float32),
        mesh=mesh,
        scratch_types=[
            pltpu.VMEM((CHUNK,), jnp.float32),
            pltpu.VMEM((CHUNK,), jnp.float32),
            pltpu.VMEM((3 * D * L,), jnp.float32),
            pltpu.VMEM((5, L), jnp.float32),
            pltpu.VMEM((L,), jnp.float32),
            pltpu.SemaphoreType.DMA,
            pltpu.SemaphoreType.DMA,
            pltpu.SemaphoreType.DMA,
        ],
    )(x_flat, coef)


def _make_coef(W, b):
    W0 = W.reshape(D, KTOP + 1)
    Wt = W0.astype(jnp.bfloat16).astype(jnp.float32)
    w_pad = jnp.concatenate([Wt, jnp.zeros_like(Wt)], axis=0)
    return jnp.concatenate(
        [w_pad.T, jnp.full((1, L), b[0], jnp.float32)], axis=0)


def _flat_view(x):
    return x.reshape(B, N // RUN, RUN, D).transpose(0, 1, 3, 2).reshape(-1)


def kernel(x, W, b):
    out = _launch(_flat_view(x), _make_coef(W, b))
    return out[:, :BPW].reshape(B, 1)
